# Initial kernel scaffold; baseline (speedup 1.0000x reference)
import jax
import jax.numpy as jnp
from jax.experimental import pallas as pl


def kernel(pos_ids, table):
    raise NotImplementedError("write your pallas kernel here")


# speedup vs baseline: 2.3197x; 2.3197x over previous
import functools

import jax
import jax.numpy as jnp
from jax import lax
from jax.experimental import pallas as pl
from jax.experimental.pallas import tpu as pltpu
from jax.experimental.pallas import tpu_sc as plsc

_D = 1024
_B = 4 * 8192
_NW = 32
_BPW = _B // _NW
_CH = 32
_NCH = _BPW // _CH

_mesh = plsc.VectorSubcoreMesh(core_axis_name="c", subcore_axis_name="s")


@functools.partial(
    pl.kernel,
    mesh=_mesh,
    out_type=jax.ShapeDtypeStruct((_B, _D), jnp.float32),
    scratch_types=[
        pltpu.VMEM((_BPW,), jnp.int32),
        pltpu.VMEM((2, _CH, _D), jnp.float32),
        pltpu.SemaphoreType.DMA,
        pltpu.SemaphoreType.DMA,
    ],
)
def _embed_gather(idx_hbm, table_hbm, out_hbm, idx_v, rows_v, gsem, osem):
    wid = lax.axis_index("s") * 2 + lax.axis_index("c")
    base = wid * _BPW
    pltpu.sync_copy(idx_hbm.at[pl.ds(base, _BPW)], idx_v)

    gathers = [None, None]
    outs = [None, None]
    gathers[0] = pltpu.async_copy(
        table_hbm.at[idx_v.at[pl.ds(0, _CH)]], rows_v.at[0], gsem)
    for g in range(_NCH):
        b = g & 1
        nb = 1 - b
        if g + 1 < _NCH:
            if outs[nb] is not None:
                outs[nb].wait()
            gathers[nb] = pltpu.async_copy(
                table_hbm.at[idx_v.at[pl.ds((g + 1) * _CH, _CH)]],
                rows_v.at[nb], gsem)
        gathers[b].wait()
        outs[b] = pltpu.async_copy(
            rows_v.at[b], out_hbm.at[pl.ds(base + g * _CH, _CH)], osem)
    outs[0].wait()
    outs[1].wait()


def kernel(pos_ids, table):
    idx = pos_ids.reshape(-1).astype(jnp.int32)
    out = _embed_gather(idx, table)
    return out.reshape(pos_ids.shape + (table.shape[1],))


# speedup vs baseline: 2.3211x; 1.0006x over previous
import functools

import jax
import jax.numpy as jnp
from jax import lax
from jax.experimental import pallas as pl
from jax.experimental.pallas import tpu as pltpu
from jax.experimental.pallas import tpu_sc as plsc

_D = 1024
_B = 4 * 8192
_NW = 32
_BPW = _B // _NW
_CH = 32
_NCH = _BPW // _CH
_NBUF = 3

_mesh = plsc.VectorSubcoreMesh(core_axis_name="c", subcore_axis_name="s")


@functools.partial(
    pl.kernel,
    mesh=_mesh,
    out_type=jax.ShapeDtypeStruct((_B, _D), jnp.float32),
    scratch_types=[
        pltpu.VMEM((_NCH, _CH), jnp.int32),
        pltpu.VMEM((_NBUF, _CH, _D), jnp.float32),
    ]
    + [pltpu.SemaphoreType.DMA] * (2 * _NBUF),
)
def _embed_gather(idx_hbm, table_hbm, out_hbm, idx_v, rows_v, *sems):
    gsems, osems = sems[:_NBUF], sems[_NBUF:]
    wid = lax.axis_index("s") * 2 + lax.axis_index("c")
    base = wid * _BPW
    pltpu.sync_copy(idx_hbm.at[pl.ds(wid * _NCH, _NCH)], idx_v)

    gathers = [None] * _NBUF
    outs = [None] * _NBUF

    def start_gather(g):
        b = g % _NBUF
        gathers[b] = pltpu.async_copy(
            table_hbm.at[idx_v.at[g]], rows_v.at[b], gsems[b])

    for g in range(_NBUF):
        start_gather(g)
    for g in range(_NCH):
        b = g % _NBUF
        gathers[b].wait()
        outs[b] = pltpu.async_copy(
            rows_v.at[b], out_hbm.at[pl.ds(base + g * _CH, _CH)], osems[b])
        if g + _NBUF < _NCH:
            outs[b].wait()
            start_gather(g + _NBUF)
    for b in range(_NBUF):
        if _NCH - 1 - b >= 0:
            outs[(_NCH - 1 - b) % _NBUF].wait()


def kernel(pos_ids, table):
    idx = pos_ids.reshape(_B // _CH, _CH).astype(jnp.int32)
    out = _embed_gather(idx, table)
    return out.reshape(pos_ids.shape + (table.shape[1],))
